# baseline (speedup 1.0000x reference)
import functools

import jax
import jax.numpy as jnp
from jax import lax
from jax.experimental import pallas as pl
from jax.experimental.pallas import tpu as pltpu
from jax.experimental.pallas import tpu_sc as plsc

BSZ = 1024
NVISITS = 50
VISIT = 26
ED = 64
ROWS = NVISITS * VISIT
NC, NS = 2, 16
NW = NC * NS
BATCH_PER_W = BSZ // NW
NPAIR = BATCH_PER_W // 2
LANES = 16
BLANES = 32
NVREG = ED // BLANES
CHUNKS = [(j * 104, 104) for j in range(12)] + [(1248, 52)]


def _embed_grid():
    mesh = plsc.VectorSubcoreMesh(core_axis_name="c", subcore_axis_name="s")

    @functools.partial(
        pl.kernel,
        mesh=mesh,
        compiler_params=pltpu.CompilerParams(
            use_tc_tiling_on_sc=False, needs_layout_passes=False),
        out_type=jax.ShapeDtypeStruct((BSZ, NVISITS, ED), jnp.float32),
        scratch_types=[
            pltpu.VMEM((2, 2, ROWS), jnp.int32),
            pltpu.VMEM((2, ROWS, ED), jnp.bfloat16),
            pltpu.VMEM((2, NVISITS, ED), jnp.float32),
            pltpu.SemaphoreType.DMA,
            pltpu.SemaphoreType.DMA,
            pltpu.SemaphoreType.DMA,
            pltpu.SemaphoreType.DMA,
            pltpu.SemaphoreType.DMA,
        ],
    )
    def body(idx_hbm, table_hbm, out_hbm, idx_v, rows_v, out_v, sem_a, sem_b,
             sem_oa, sem_ob, sem_i):
        wid = lax.axis_index("s") * NC + lax.axis_index("c")
        base_b = wid * BATCH_PER_W
        even = 2 * lax.broadcasted_iota(jnp.int32, (LANES,), 0)
        odd = even + 1

        def stage_desc(k, kp):
            return pltpu.make_async_copy(
                idx_hbm.at[pl.ds(base_b + 2 * k, 2)], idx_v.at[kp], sem_i)

        def fire(kp, bb, g, sem):
            for off, ln in CHUNKS:
                pltpu.async_copy(
                    table_hbm.at[idx_v.at[kp, bb, pl.ds(off, ln)]],
                    rows_v.at[g, pl.ds(off, ln)],
                    sem,
                )

        def drain(g, sem):
            pltpu.make_async_copy(
                table_hbm.at[pl.ds(0, ROWS)], rows_v.at[g], sem).wait()

        def out_desc(g, sem):
            return pltpu.make_async_copy(out_v.at[g], out_hbm.at[base_b], sem)

        def compute_store(b, g, sem):
            def seg_body(s, c2):
                r0 = s * VISIT
                for cc in range(NVREG):
                    sl = pl.ds(cc * BLANES, BLANES)
                    vals = [rows_v[g, r0 + r, sl] for r in range(VISIT)]
                    while len(vals) > 1:
                        nxt = [vals[k] + vals[k + 1]
                               for k in range(0, len(vals) - 1, 2)]
                        if len(vals) % 2:
                            nxt[-1] = nxt[-1] + vals[-1]
                        vals = nxt
                    lo, hi = plsc.unpack(
                        vals[0], format=plsc.PackFormat.INTERLEAVED)
                    orow = out_v.at[g, s, pl.ds(cc * BLANES, BLANES)]
                    plsc.store_scatter(orow, [even], lo)
                    plsc.store_scatter(orow, [odd], hi)
                return c2

            lax.fori_loop(0, NVISITS, seg_body, 0)
            pltpu.async_copy(out_v.at[g], out_hbm.at[b], sem)

        d0 = stage_desc(0, 0)
        d0.start()
        d0.wait()
        fire(0, 0, 0, sem_a)

        def pair_body(k, carry):
            kp = lax.rem(k, 2)
            b0 = base_b + 2 * k
            fire(kp, 1, 1, sem_b)

            @pl.when(k + 1 < NPAIR)
            def _():
                stage_desc(k + 1, 1 - kp).start()

            drain(0, sem_a)

            @pl.when(k > 0)
            def _():
                out_desc(0, sem_oa).wait()

            compute_store(b0, 0, sem_oa)

            @pl.when(k + 1 < NPAIR)
            def _():
                stage_desc(k + 1, 1 - kp).wait()
                fire(1 - kp, 0, 0, sem_a)

            drain(1, sem_b)

            @pl.when(k > 0)
            def _():
                out_desc(1, sem_ob).wait()

            compute_store(b0 + 1, 1, sem_ob)
            return carry

        lax.fori_loop(0, NPAIR, pair_body, 0)
        out_desc(0, sem_oa).wait()
        out_desc(1, sem_ob).wait()

    return body


_EMBED = _embed_grid()


def kernel(visit_tensor, table):
    return _EMBED(visit_tensor.reshape(BSZ, ROWS), table.astype(jnp.bfloat16))
